# baseline (speedup 1.0000x reference)
import functools

import jax
import jax.numpy as jnp
from jax import lax
from jax.experimental import pallas as pl
from jax.experimental.pallas import tpu as pltpu
from jax.experimental.pallas import tpu_sc as plsc

N = 10000
E = 320000
D = 128

NUM_TILES = 32
N_PAD = 10240
N_ACC = 10112
ACC_PER_SUB = N_ACC // 16
DEG_PER_SUB = N_PAD // 16
E_PAD = NUM_TILES * 10240
EDGES_PER_TILE = E_PAD // NUM_TILES
CHUNK = 128
NCHUNKS = EDGES_PER_TILE // CHUNK
NB = 3
NC0 = 128
NC1 = 32
ZROWS = 128

_mesh = plsc.VectorSubcoreMesh(core_axis_name="c", subcore_axis_name="s")


@functools.partial(
    pl.kernel,
    out_type=jax.ShapeDtypeStruct((2, N_PAD), jnp.float32),
    mesh=_mesh,
    scratch_types=[
        pltpu.VMEM((NCHUNKS, CHUNK), jnp.int32),
        pltpu.VMEM((CHUNK,), jnp.float32),
        pltpu.VMEM((DEG_PER_SUB,), jnp.float32),
        pltpu.VMEM_SHARED((N_PAD,), jnp.float32),
        pltpu.SemaphoreType.DMA,
    ],
)
def _deg_kernel(dst_hbm, degpart_hbm, idx_v, ones_v, zbuf_v, acc_sh, sem):
    c = lax.axis_index("c")
    s = lax.axis_index("s")
    wid = s * 2 + c

    pltpu.sync_copy(dst_hbm.at[pl.ds(wid * NCHUNKS, NCHUNKS)], idx_v)

    def _fill(i, _):
        zbuf_v[pl.ds(i * 16, 16)] = jnp.zeros((16,), jnp.float32)
        return 0

    lax.fori_loop(0, DEG_PER_SUB // 16, _fill, 0)

    def _fill1(i, _):
        ones_v[pl.ds(i * 16, 16)] = jnp.ones((16,), jnp.float32)
        return 0

    lax.fori_loop(0, CHUNK // 16, _fill1, 0)

    pltpu.sync_copy(zbuf_v, acc_sh.at[pl.ds(s * DEG_PER_SUB, DEG_PER_SUB)])
    plsc.subcore_barrier()

    def _fire(j, _):
        pltpu.async_copy(ones_v, acc_sh.at[idx_v.at[j]], sem, add=True)
        return 0

    lax.fori_loop(0, NCHUNKS, _fire, 0)

    def _drain(j, _):
        pltpu.make_async_copy(ones_v, acc_sh.at[idx_v.at[0]], sem).wait()
        return 0

    lax.fori_loop(0, NCHUNKS, _drain, 0)
    plsc.subcore_barrier()

    pltpu.sync_copy(
        acc_sh.at[pl.ds(s * DEG_PER_SUB, DEG_PER_SUB)],
        degpart_hbm.at[c, pl.ds(s * DEG_PER_SUB, DEG_PER_SUB)],
    )


@functools.partial(
    pl.kernel,
    out_type=jax.ShapeDtypeStruct((2, N_PAD, D), jnp.float32),
    mesh=_mesh,
    scratch_types=[
        pltpu.VMEM((NB, CHUNK), jnp.int32),
        pltpu.VMEM((NB, CHUNK), jnp.int32),
        pltpu.VMEM((NB, CHUNK, D), jnp.float32),
        pltpu.VMEM_SHARED((N_ACC, D), jnp.float32),
        pltpu.SemaphoreType.DMA,
        pltpu.SemaphoreType.DMA,
        pltpu.SemaphoreType.DMA,
        pltpu.SemaphoreType.DMA,
        pltpu.SemaphoreType.DMA,
        pltpu.SemaphoreType.DMA,
        pltpu.SemaphoreType.DMA,
        pltpu.SemaphoreType.DMA,
        pltpu.SemaphoreType.DMA,
    ],
)
def _prop_kernel(y_hbm, src_hbm, dst_hbm, part_hbm,
                 sidx_v, didx_v, rows_v, acc_sh,
                 g0, g1, g2, ls0, ls1, ls2, ld0, ld1, ld2):
    gsems = (g0, g1, g2)
    lssems = (ls0, ls1, ls2)
    ldsems = (ld0, ld1, ld2)
    c = lax.axis_index("c")
    s = lax.axis_index("s")
    crow = jnp.where(c == 0, s * NC0, 16 * NC0 + s * NC1)
    cnt = jnp.where(c == 0, NC0, NC1)

    def _fill(i, _):
        r = i // (D // 16)
        k = i % (D // 16)
        rows_v[0, r, pl.ds(k * 16, 16)] = jnp.zeros((16,), jnp.float32)
        return 0

    lax.fori_loop(0, ZROWS * (D // 16), _fill, 0)

    zbase = s * ACC_PER_SUB

    def _zero(t, _):
        pltpu.sync_copy(
            rows_v.at[0], acc_sh.at[pl.ds(zbase + t * ZROWS, ZROWS)])
        return 0

    lax.fori_loop(0, ACC_PER_SUB // ZROWS, _zero, 0)
    pltpu.sync_copy(
        rows_v.at[0, pl.ds(0, ACC_PER_SUB % ZROWS)],
        acc_sh.at[pl.ds(zbase + (ACC_PER_SUB // ZROWS) * ZROWS,
                        ACC_PER_SUB % ZROWS)],
    )
    plsc.subcore_barrier()

    def _fire_sload(j, b):
        pltpu.async_copy(
            src_hbm.at[pl.ds(crow + j, 1)], sidx_v.at[pl.ds(b, 1)], lssems[b])

    def _wait_sload(b):
        pltpu.make_async_copy(
            src_hbm.at[pl.ds(crow, 1)], sidx_v.at[pl.ds(b, 1)],
            lssems[b]).wait()

    def _fire_dload(j, b):
        pltpu.async_copy(
            dst_hbm.at[pl.ds(crow + j, 1)], didx_v.at[pl.ds(b, 1)], ldsems[b])

    def _wait_dload(b):
        pltpu.make_async_copy(
            dst_hbm.at[pl.ds(crow, 1)], didx_v.at[pl.ds(b, 1)],
            ldsems[b]).wait()

    def _fire_gather(j, b):
        pltpu.async_copy(y_hbm.at[sidx_v.at[b]], rows_v.at[b], gsems[b])

    def _wait_gather(b):
        pltpu.make_async_copy(
            y_hbm.at[sidx_v.at[0]], rows_v.at[b], gsems[b]).wait()

    for b in range(NB):
        _fire_sload(b, b)
        _fire_dload(b, b)
    for b in range(NB):
        _wait_sload(b)
        _fire_gather(b, b)

    def _body(t, _):
        for b in range(NB):
            j = t * NB + b

            @pl.when(j < cnt)
            def _():
                _wait_gather(b)

                @pl.when(j + NB < cnt)
                def _():
                    _fire_sload(j + NB, b)

                _wait_dload(b)
                pltpu.sync_copy(
                    rows_v.at[b], acc_sh.at[didx_v.at[b]], add=True)

                @pl.when(j + NB < cnt)
                def _():
                    _fire_dload(j + NB, b)
                    _wait_sload(b)
                    _fire_gather(j + NB, b)

        return 0

    lax.fori_loop(0, (cnt + NB - 1) // NB, _body, 0)
    plsc.subcore_barrier()

    pltpu.sync_copy(
        acc_sh.at[pl.ds(s * ACC_PER_SUB, ACC_PER_SUB)],
        part_hbm.at[c, pl.ds(s * ACC_PER_SUB, ACC_PER_SUB)],
    )


RB = 1024


def _dinv_from(degp_ref):
    deg = degp_ref[0, :] + degp_ref[1, :] + 1.0
    return lax.rsqrt(deg)


def _tcA_body(x_ref, w_ref, degp_ref, y_ref):
    dinv = _dinv_from(degp_ref)
    xw = jnp.dot(x_ref[...], w_ref[...], preferred_element_type=jnp.float32)
    rows = pl.program_id(0) * RB + lax.broadcasted_iota(jnp.int32, (RB, 1), 0)
    y_ref[...] = jnp.where(rows < N, xw * dinv[:, None], 0.0)


def _tcB_body(p_ref, y1_ref, degp_ref, b_ref, w_ref, y2_ref):
    dinv = _dinv_from(degp_ref)
    pre = (p_ref[0] + p_ref[1] + y1_ref[...]) * dinv[:, None] + b_ref[...]
    h = jnp.where(pre > 0, pre, jnp.exp(jnp.minimum(pre, 0.0)) - 1.0)
    hw = jnp.dot(h, w_ref[...], preferred_element_type=jnp.float32)
    rows = pl.program_id(0) * RB + lax.broadcasted_iota(jnp.int32, (RB, 1), 0)
    y2_ref[...] = jnp.where(rows < N, hw * dinv[:, None], 0.0)


def _tcC_body(p_ref, y2_ref, degp_ref, b_ref, out_ref):
    dinv = _dinv_from(degp_ref)
    out_ref[...] = (
        (p_ref[0] + p_ref[1] + y2_ref[...]) * dinv[:, None] + b_ref[...])


_row_spec = pl.BlockSpec((RB, D), lambda i: (i, 0))
_mat_spec = pl.BlockSpec((D, D), lambda i: (0, 0))
_deg_spec = pl.BlockSpec((2, RB), lambda i: (0, i))
_part_spec = pl.BlockSpec((2, RB, D), lambda i: (0, i, 0))
_bias_spec = pl.BlockSpec((1, D), lambda i: (0, 0))
_grid = (N_PAD // RB,)

_tcA = pl.pallas_call(
    _tcA_body,
    grid=_grid,
    in_specs=[_row_spec, _mat_spec, _deg_spec],
    out_specs=_row_spec,
    out_shape=jax.ShapeDtypeStruct((N_PAD, D), jnp.float32),
)

_tcB = pl.pallas_call(
    _tcB_body,
    grid=_grid,
    in_specs=[_part_spec, _row_spec, _deg_spec, _bias_spec, _mat_spec],
    out_specs=_row_spec,
    out_shape=jax.ShapeDtypeStruct((N_PAD, D), jnp.float32),
)

_tcC = pl.pallas_call(
    _tcC_body,
    grid=_grid,
    in_specs=[_part_spec, _row_spec, _deg_spec, _bias_spec],
    out_specs=_row_spec,
    out_shape=jax.ShapeDtypeStruct((N_PAD, D), jnp.float32),
)


def kernel(x, edge_index, W1, b1, W2, b2):
    src = edge_index[0].astype(jnp.int32)
    dst = edge_index[1].astype(jnp.int32)
    pad = jnp.full((E_PAD - E,), N, jnp.int32)
    src_p = jnp.concatenate([src, pad]).reshape(E_PAD // CHUNK, CHUNK)
    dst_p = jnp.concatenate([dst, pad]).reshape(E_PAD // CHUNK, CHUNK)
    x_p = jnp.pad(x, ((0, N_PAD - N), (0, 0)))
    b1r = b1.reshape(1, D)
    b2r = b2.reshape(1, D)

    degp = _deg_kernel(dst_p)
    y1 = _tcA(x_p, W1, degp)
    p1 = _prop_kernel(y1, src_p, dst_p)
    y2 = _tcB(p1, y1, degp, b1r, W2)
    p2 = _prop_kernel(y2, src_p, dst_p)
    out = _tcC(p2, y2, degp, b2r)
    return out[:N]


# speedup vs baseline: 1.0541x; 1.0541x over previous
import functools

import jax
import jax.numpy as jnp
from jax import lax
from jax.experimental import pallas as pl
from jax.experimental.pallas import tpu as pltpu
from jax.experimental.pallas import tpu_sc as plsc

N = 10000
E = 320000
D = 128

NUM_TILES = 32
N_PAD = 10240
N_ACC = 10112
ACC_PER_SUB = N_ACC // 16
DEG_PER_SUB = N_PAD // 16
E_PAD = NUM_TILES * 10240
EDGES_PER_TILE = E_PAD // NUM_TILES
CHUNK = 128
NCHUNKS = EDGES_PER_TILE // CHUNK
NB = 3
NC0 = 136
NC1 = 24
ZROWS = 128

_mesh = plsc.VectorSubcoreMesh(core_axis_name="c", subcore_axis_name="s")


@functools.partial(
    pl.kernel,
    out_type=jax.ShapeDtypeStruct((2, N_PAD), jnp.float32),
    mesh=_mesh,
    scratch_types=[
        pltpu.VMEM((NCHUNKS, CHUNK), jnp.int32),
        pltpu.VMEM((CHUNK,), jnp.float32),
        pltpu.VMEM((DEG_PER_SUB,), jnp.float32),
        pltpu.VMEM_SHARED((N_PAD,), jnp.float32),
        pltpu.SemaphoreType.DMA,
    ],
)
def _deg_kernel(dst_hbm, degpart_hbm, idx_v, ones_v, zbuf_v, acc_sh, sem):
    c = lax.axis_index("c")
    s = lax.axis_index("s")
    wid = s * 2 + c

    pltpu.sync_copy(dst_hbm.at[pl.ds(wid * NCHUNKS, NCHUNKS)], idx_v)

    def _fill(i, _):
        zbuf_v[pl.ds(i * 16, 16)] = jnp.zeros((16,), jnp.float32)
        return 0

    lax.fori_loop(0, DEG_PER_SUB // 16, _fill, 0)

    def _fill1(i, _):
        ones_v[pl.ds(i * 16, 16)] = jnp.ones((16,), jnp.float32)
        return 0

    lax.fori_loop(0, CHUNK // 16, _fill1, 0)

    pltpu.sync_copy(zbuf_v, acc_sh.at[pl.ds(s * DEG_PER_SUB, DEG_PER_SUB)])
    plsc.subcore_barrier()

    def _fire(j, _):
        pltpu.async_copy(ones_v, acc_sh.at[idx_v.at[j]], sem, add=True)
        return 0

    lax.fori_loop(0, NCHUNKS, _fire, 0)

    def _drain(j, _):
        pltpu.make_async_copy(ones_v, acc_sh.at[idx_v.at[0]], sem).wait()
        return 0

    lax.fori_loop(0, NCHUNKS, _drain, 0)
    plsc.subcore_barrier()

    pltpu.sync_copy(
        acc_sh.at[pl.ds(s * DEG_PER_SUB, DEG_PER_SUB)],
        degpart_hbm.at[c, pl.ds(s * DEG_PER_SUB, DEG_PER_SUB)],
    )


@functools.partial(
    pl.kernel,
    out_type=jax.ShapeDtypeStruct((2, N_PAD, D), jnp.float32),
    mesh=_mesh,
    scratch_types=[
        pltpu.VMEM((NB, CHUNK), jnp.int32),
        pltpu.VMEM((NB, CHUNK), jnp.int32),
        pltpu.VMEM((NB, CHUNK, D), jnp.float32),
        pltpu.VMEM_SHARED((N_ACC, D), jnp.float32),
        pltpu.SemaphoreType.DMA,
        pltpu.SemaphoreType.DMA,
        pltpu.SemaphoreType.DMA,
        pltpu.SemaphoreType.DMA,
        pltpu.SemaphoreType.DMA,
        pltpu.SemaphoreType.DMA,
        pltpu.SemaphoreType.DMA,
        pltpu.SemaphoreType.DMA,
        pltpu.SemaphoreType.DMA,
    ],
)
def _prop_kernel(y_hbm, src_hbm, dst_hbm, part_hbm,
                 sidx_v, didx_v, rows_v, acc_sh,
                 g0, g1, g2, ls0, ls1, ls2, ld0, ld1, ld2):
    gsems = (g0, g1, g2)
    lssems = (ls0, ls1, ls2)
    ldsems = (ld0, ld1, ld2)
    c = lax.axis_index("c")
    s = lax.axis_index("s")
    crow = jnp.where(c == 0, s * NC0, 16 * NC0 + s * NC1)
    cnt = jnp.where(c == 0, NC0, NC1)

    def _fill(i, _):
        r = i // (D // 16)
        k = i % (D // 16)
        rows_v[0, r, pl.ds(k * 16, 16)] = jnp.zeros((16,), jnp.float32)
        return 0

    lax.fori_loop(0, ZROWS * (D // 16), _fill, 0)

    zbase = s * ACC_PER_SUB

    def _zero(t, _):
        pltpu.sync_copy(
            rows_v.at[0], acc_sh.at[pl.ds(zbase + t * ZROWS, ZROWS)])
        return 0

    lax.fori_loop(0, ACC_PER_SUB // ZROWS, _zero, 0)
    pltpu.sync_copy(
        rows_v.at[0, pl.ds(0, ACC_PER_SUB % ZROWS)],
        acc_sh.at[pl.ds(zbase + (ACC_PER_SUB // ZROWS) * ZROWS,
                        ACC_PER_SUB % ZROWS)],
    )
    plsc.subcore_barrier()

    def _fire_sload(j, b):
        pltpu.async_copy(
            src_hbm.at[pl.ds(crow + j, 1)], sidx_v.at[pl.ds(b, 1)], lssems[b])

    def _wait_sload(b):
        pltpu.make_async_copy(
            src_hbm.at[pl.ds(crow, 1)], sidx_v.at[pl.ds(b, 1)],
            lssems[b]).wait()

    def _fire_dload(j, b):
        pltpu.async_copy(
            dst_hbm.at[pl.ds(crow + j, 1)], didx_v.at[pl.ds(b, 1)], ldsems[b])

    def _wait_dload(b):
        pltpu.make_async_copy(
            dst_hbm.at[pl.ds(crow, 1)], didx_v.at[pl.ds(b, 1)],
            ldsems[b]).wait()

    def _fire_gather(j, b):
        pltpu.async_copy(y_hbm.at[sidx_v.at[b]], rows_v.at[b], gsems[b])

    def _wait_gather(b):
        pltpu.make_async_copy(
            y_hbm.at[sidx_v.at[0]], rows_v.at[b], gsems[b]).wait()

    for b in range(NB):
        _fire_sload(b, b)
        _fire_dload(b, b)
    for b in range(NB):
        _wait_sload(b)
        _fire_gather(b, b)

    def _body(t, _):
        for b in range(NB):
            j = t * NB + b

            @pl.when(j < cnt)
            def _():
                _wait_gather(b)

                @pl.when(j + NB < cnt)
                def _():
                    _fire_sload(j + NB, b)

                _wait_dload(b)
                pltpu.sync_copy(
                    rows_v.at[b], acc_sh.at[didx_v.at[b]], add=True)

                @pl.when(j + NB < cnt)
                def _():
                    _fire_dload(j + NB, b)
                    _wait_sload(b)
                    _fire_gather(j + NB, b)

        return 0

    lax.fori_loop(0, (cnt + NB - 1) // NB, _body, 0)
    plsc.subcore_barrier()

    pltpu.sync_copy(
        acc_sh.at[pl.ds(s * ACC_PER_SUB, ACC_PER_SUB)],
        part_hbm.at[c, pl.ds(s * ACC_PER_SUB, ACC_PER_SUB)],
    )


RB = 1024


def _dinv_from(degp_ref):
    deg = degp_ref[0, :] + degp_ref[1, :] + 1.0
    return lax.rsqrt(deg)


def _tcA_body(x_ref, w_ref, degp_ref, y_ref):
    dinv = _dinv_from(degp_ref)
    xw = jnp.dot(x_ref[...], w_ref[...], preferred_element_type=jnp.float32)
    rows = pl.program_id(0) * RB + lax.broadcasted_iota(jnp.int32, (RB, 1), 0)
    y_ref[...] = jnp.where(rows < N, xw * dinv[:, None], 0.0)


def _tcB_body(p_ref, y1_ref, degp_ref, b_ref, w_ref, y2_ref):
    dinv = _dinv_from(degp_ref)
    pre = (p_ref[0] + p_ref[1] + y1_ref[...]) * dinv[:, None] + b_ref[...]
    h = jnp.where(pre > 0, pre, jnp.exp(jnp.minimum(pre, 0.0)) - 1.0)
    hw = jnp.dot(h, w_ref[...], preferred_element_type=jnp.float32)
    rows = pl.program_id(0) * RB + lax.broadcasted_iota(jnp.int32, (RB, 1), 0)
    y2_ref[...] = jnp.where(rows < N, hw * dinv[:, None], 0.0)


def _tcC_body(p_ref, y2_ref, degp_ref, b_ref, out_ref):
    dinv = _dinv_from(degp_ref)
    out_ref[...] = (
        (p_ref[0] + p_ref[1] + y2_ref[...]) * dinv[:, None] + b_ref[...])


_row_spec = pl.BlockSpec((RB, D), lambda i: (i, 0))
_mat_spec = pl.BlockSpec((D, D), lambda i: (0, 0))
_deg_spec = pl.BlockSpec((2, RB), lambda i: (0, i))
_part_spec = pl.BlockSpec((2, RB, D), lambda i: (0, i, 0))
_bias_spec = pl.BlockSpec((1, D), lambda i: (0, 0))
_grid = (N_PAD // RB,)

_tcA = pl.pallas_call(
    _tcA_body,
    grid=_grid,
    in_specs=[_row_spec, _mat_spec, _deg_spec],
    out_specs=_row_spec,
    out_shape=jax.ShapeDtypeStruct((N_PAD, D), jnp.float32),
)

_tcB = pl.pallas_call(
    _tcB_body,
    grid=_grid,
    in_specs=[_part_spec, _row_spec, _deg_spec, _bias_spec, _mat_spec],
    out_specs=_row_spec,
    out_shape=jax.ShapeDtypeStruct((N_PAD, D), jnp.float32),
)

_tcC = pl.pallas_call(
    _tcC_body,
    grid=_grid,
    in_specs=[_part_spec, _row_spec, _deg_spec, _bias_spec],
    out_specs=_row_spec,
    out_shape=jax.ShapeDtypeStruct((N_PAD, D), jnp.float32),
)


def kernel(x, edge_index, W1, b1, W2, b2):
    src = edge_index[0].astype(jnp.int32)
    dst = edge_index[1].astype(jnp.int32)
    pad = jnp.full((E_PAD - E,), N, jnp.int32)
    src_p = jnp.concatenate([src, pad]).reshape(E_PAD // CHUNK, CHUNK)
    dst_p = jnp.concatenate([dst, pad]).reshape(E_PAD // CHUNK, CHUNK)
    x_p = jnp.pad(x, ((0, N_PAD - N), (0, 0)))
    b1r = b1.reshape(1, D)
    b2r = b2.reshape(1, D)

    degp = _deg_kernel(dst_p)
    y1 = _tcA(x_p, W1, degp)
    p1 = _prop_kernel(y1, src_p, dst_p)
    y2 = _tcB(p1, y1, degp, b1r, W2)
    p2 = _prop_kernel(y2, src_p, dst_p)
    out = _tcC(p2, y2, degp, b2r)
    return out[:N]


# speedup vs baseline: 4.1341x; 3.9219x over previous
import functools

import jax
import jax.numpy as jnp
from jax import lax
from jax.experimental import pallas as pl
from jax.experimental.pallas import tpu as pltpu
from jax.experimental.pallas import tpu_sc as plsc

N = 10000
E = 320000
D = 128

NUM_TILES = 32
N_PAD = 10240
N_ACC = 10112
ACC_PER_SUB = N_ACC // 16
DEG_PER_SUB = N_PAD // 16
E_PAD = NUM_TILES * 10240
EDGES_PER_TILE = E_PAD // NUM_TILES
CHUNK = 128
NCHUNKS = EDGES_PER_TILE // CHUNK
NB = 3
NC0 = 80
NC1 = 80
ZROWS = 128

_mesh = plsc.VectorSubcoreMesh(core_axis_name="c", subcore_axis_name="s")


@functools.partial(
    pl.kernel,
    out_type=jax.ShapeDtypeStruct((2, N_PAD), jnp.float32),
    mesh=_mesh,
    scratch_types=[
        pltpu.VMEM((NCHUNKS, CHUNK), jnp.int32),
        pltpu.VMEM((CHUNK,), jnp.float32),
        pltpu.VMEM((DEG_PER_SUB,), jnp.float32),
        pltpu.VMEM_SHARED((N_PAD,), jnp.float32),
        pltpu.SemaphoreType.DMA,
    ],
)
def _deg_kernel(dst_hbm, degpart_hbm, idx_v, ones_v, zbuf_v, acc_sh, sem):
    c = lax.axis_index("c")
    s = lax.axis_index("s")
    wid = s * 2 + c

    pltpu.sync_copy(dst_hbm.at[pl.ds(wid * NCHUNKS, NCHUNKS)], idx_v)

    def _fill(i, _):
        zbuf_v[pl.ds(i * 16, 16)] = jnp.zeros((16,), jnp.float32)
        return 0

    lax.fori_loop(0, DEG_PER_SUB // 16, _fill, 0)

    def _fill1(i, _):
        ones_v[pl.ds(i * 16, 16)] = jnp.ones((16,), jnp.float32)
        return 0

    lax.fori_loop(0, CHUNK // 16, _fill1, 0)

    pltpu.sync_copy(zbuf_v, acc_sh.at[pl.ds(s * DEG_PER_SUB, DEG_PER_SUB)])
    plsc.subcore_barrier()

    def _fire(j, _):
        pltpu.async_copy(ones_v, acc_sh.at[idx_v.at[j]], sem, add=True)
        return 0

    lax.fori_loop(0, NCHUNKS, _fire, 0)

    def _drain(j, _):
        pltpu.make_async_copy(ones_v, acc_sh.at[idx_v.at[0]], sem).wait()
        return 0

    lax.fori_loop(0, NCHUNKS, _drain, 0)
    plsc.subcore_barrier()

    pltpu.sync_copy(
        acc_sh.at[pl.ds(s * DEG_PER_SUB, DEG_PER_SUB)],
        degpart_hbm.at[c, pl.ds(s * DEG_PER_SUB, DEG_PER_SUB)],
    )


@functools.partial(
    pl.kernel,
    out_type=jax.ShapeDtypeStruct((2, N_PAD, D), jnp.float32),
    mesh=_mesh,
    scratch_types=[
        pltpu.VMEM((NB, CHUNK), jnp.int32),
        pltpu.VMEM((NB, CHUNK), jnp.int32),
        pltpu.VMEM((NB, CHUNK, D), jnp.float32),
        pltpu.VMEM_SHARED((N_ACC, D), jnp.float32),
        pltpu.SemaphoreType.DMA,
        pltpu.SemaphoreType.DMA,
        pltpu.SemaphoreType.DMA,
        pltpu.SemaphoreType.DMA,
        pltpu.SemaphoreType.DMA,
        pltpu.SemaphoreType.DMA,
        pltpu.SemaphoreType.DMA,
        pltpu.SemaphoreType.DMA,
        pltpu.SemaphoreType.DMA,
    ],
)
def _prop_kernel(y_hbm, src_hbm, dst_hbm, part_hbm,
                 sidx_v, didx_v, rows_v, acc_sh,
                 g0, g1, g2, ls0, ls1, ls2, ld0, ld1, ld2):
    gsems = (g0, g1, g2)
    lssems = (ls0, ls1, ls2)
    ldsems = (ld0, ld1, ld2)
    c = lax.axis_index("c")
    s = lax.axis_index("s")
    crow = jnp.where(c == 0, s * NC0, 16 * NC0 + s * NC1)
    cnt = jnp.where(c == 0, NC0, NC1)

    def _fill(i, _):
        r = i // (D // 16)
        k = i % (D // 16)
        rows_v[0, r, pl.ds(k * 16, 16)] = jnp.zeros((16,), jnp.float32)
        return 0

    lax.fori_loop(0, ZROWS * (D // 16), _fill, 0)

    zbase = s * ACC_PER_SUB

    def _zero(t, _):
        pltpu.sync_copy(
            rows_v.at[0], acc_sh.at[pl.ds(zbase + t * ZROWS, ZROWS)])
        return 0

    lax.fori_loop(0, ACC_PER_SUB // ZROWS, _zero, 0)
    pltpu.sync_copy(
        rows_v.at[0, pl.ds(0, ACC_PER_SUB % ZROWS)],
        acc_sh.at[pl.ds(zbase + (ACC_PER_SUB // ZROWS) * ZROWS,
                        ACC_PER_SUB % ZROWS)],
    )
    plsc.subcore_barrier()

    def _fire_sload(j, b):
        pltpu.async_copy(
            src_hbm.at[pl.ds(crow + j, 1)], sidx_v.at[pl.ds(b, 1)], lssems[b])

    def _wait_sload(b):
        pltpu.make_async_copy(
            src_hbm.at[pl.ds(crow, 1)], sidx_v.at[pl.ds(b, 1)],
            lssems[b]).wait()

    def _fire_dload(j, b):
        pltpu.async_copy(
            dst_hbm.at[pl.ds(crow + j, 1)], didx_v.at[pl.ds(b, 1)], ldsems[b])

    def _wait_dload(b):
        pltpu.make_async_copy(
            dst_hbm.at[pl.ds(crow, 1)], didx_v.at[pl.ds(b, 1)],
            ldsems[b]).wait()

    def _fire_gather(j, b):
        pltpu.async_copy(y_hbm.at[sidx_v.at[b]], rows_v.at[b], gsems[b])

    def _wait_gather(b):
        pltpu.make_async_copy(
            y_hbm.at[sidx_v.at[0]], rows_v.at[b], gsems[b]).wait()

    for b in range(NB):
        _fire_sload(b, b)
        _fire_dload(b, b)
    for b in range(NB):
        _wait_sload(b)
        _fire_gather(b, b)

    def _body(t, _):
        for b in range(NB):
            j = t * NB + b

            @pl.when(j < cnt)
            def _():
                _wait_gather(b)

                @pl.when(j + NB < cnt)
                def _():
                    _fire_sload(j + NB, b)

                _wait_dload(b)
                pltpu.sync_copy(
                    rows_v.at[b], acc_sh.at[didx_v.at[b]], add=True)

                @pl.when(j + NB < cnt)
                def _():
                    _fire_dload(j + NB, b)
                    _wait_sload(b)
                    _fire_gather(j + NB, b)

        return 0

    lax.fori_loop(0, (cnt + NB - 1) // NB, _body, 0)
    plsc.subcore_barrier()

    pltpu.sync_copy(
        acc_sh.at[pl.ds(s * ACC_PER_SUB, ACC_PER_SUB)],
        part_hbm.at[c, pl.ds(s * ACC_PER_SUB, ACC_PER_SUB)],
    )


RB = 1024


def _dinv_from(degp_ref):
    deg = degp_ref[0, :] + degp_ref[1, :] + 1.0
    return lax.rsqrt(deg)


def _tcA_body(x_ref, w_ref, degp_ref, y_ref):
    dinv = _dinv_from(degp_ref)
    xw = jnp.dot(x_ref[...], w_ref[...], preferred_element_type=jnp.float32)
    rows = pl.program_id(0) * RB + lax.broadcasted_iota(jnp.int32, (RB, 1), 0)
    y_ref[...] = jnp.where(rows < N, xw * dinv[:, None], 0.0)


def _tcB_body(p_ref, y1_ref, degp_ref, b_ref, w_ref, y2_ref):
    dinv = _dinv_from(degp_ref)
    pre = (p_ref[0] + p_ref[1] + y1_ref[...]) * dinv[:, None] + b_ref[...]
    h = jnp.where(pre > 0, pre, jnp.exp(jnp.minimum(pre, 0.0)) - 1.0)
    hw = jnp.dot(h, w_ref[...], preferred_element_type=jnp.float32)
    rows = pl.program_id(0) * RB + lax.broadcasted_iota(jnp.int32, (RB, 1), 0)
    y2_ref[...] = jnp.where(rows < N, hw * dinv[:, None], 0.0)


def _tcC_body(p_ref, y2_ref, degp_ref, b_ref, out_ref):
    dinv = _dinv_from(degp_ref)
    out_ref[...] = (
        (p_ref[0] + p_ref[1] + y2_ref[...]) * dinv[:, None] + b_ref[...])


_row_spec = pl.BlockSpec((RB, D), lambda i: (i, 0))
_mat_spec = pl.BlockSpec((D, D), lambda i: (0, 0))
_deg_spec = pl.BlockSpec((2, RB), lambda i: (0, i))
_part_spec = pl.BlockSpec((2, RB, D), lambda i: (0, i, 0))
_bias_spec = pl.BlockSpec((1, D), lambda i: (0, 0))
_grid = (N_PAD // RB,)

_tcA = pl.pallas_call(
    _tcA_body,
    grid=_grid,
    in_specs=[_row_spec, _mat_spec, _deg_spec],
    out_specs=_row_spec,
    out_shape=jax.ShapeDtypeStruct((N_PAD, D), jnp.float32),
)

_tcB = pl.pallas_call(
    _tcB_body,
    grid=_grid,
    in_specs=[_part_spec, _row_spec, _deg_spec, _bias_spec, _mat_spec],
    out_specs=_row_spec,
    out_shape=jax.ShapeDtypeStruct((N_PAD, D), jnp.float32),
)

_tcC = pl.pallas_call(
    _tcC_body,
    grid=_grid,
    in_specs=[_part_spec, _row_spec, _deg_spec, _bias_spec],
    out_specs=_row_spec,
    out_shape=jax.ShapeDtypeStruct((N_PAD, D), jnp.float32),
)


def kernel(x, edge_index, W1, b1, W2, b2):
    src = edge_index[0].astype(jnp.int32)
    dst = edge_index[1].astype(jnp.int32)
    pad = N + (jnp.arange(E_PAD - E, dtype=jnp.int32) % (N_ACC - N))
    src_p = jnp.concatenate([src, pad]).reshape(E_PAD // CHUNK, CHUNK)
    dst_p = jnp.concatenate([dst, pad]).reshape(E_PAD // CHUNK, CHUNK)
    x_p = jnp.pad(x, ((0, N_PAD - N), (0, 0)))
    b1r = b1.reshape(1, D)
    b2r = b2.reshape(1, D)

    degp = _deg_kernel(dst_p)
    y1 = _tcA(x_p, W1, degp)
    p1 = _prop_kernel(y1, src_p, dst_p)
    y2 = _tcB(p1, y1, degp, b1r, W2)
    p2 = _prop_kernel(y2, src_p, dst_p)
    out = _tcC(p2, y2, degp, b2r)
    return out[:N]
